# Initial kernel scaffold; baseline (speedup 1.0000x reference)
import jax
import jax.numpy as jnp
from jax.experimental import pallas as pl


def kernel(x, W0, Wa0, g0, be0, out0W, out0b, W1, Wa1, g1, be1, out1W, out1b, projW, projb, Wih, Whh, bih, bhh, edge_index, graph_ids):
    raise NotImplementedError("write your pallas kernel here")


# speedup vs baseline: 3.7976x; 3.7976x over previous
import functools

import jax
import jax.numpy as jnp
from jax import lax
from jax.experimental import pallas as pl
from jax.experimental.pallas import tpu as pltpu
from jax.experimental.pallas import tpu_sc as plsc

N = 10000
E = 320000
B = 256
F = 74
H = 128
HEADS = 2

NPAD = 10240
NW = 32
CPW = 80
EPW = CPW * 128
EPAD = NW * EPW
TPW = NPAD // NW

_f32 = jnp.float32
_i32 = jnp.int32


BLK = 2048
NB = NPAD // BLK


def _one_hot(gid):
    return (gid == lax.broadcasted_iota(_i32, (1, B), 1)).astype(_f32)


def _bspec(shape, blocked_dim0=False):
    if blocked_dim0:
        nd = len(shape)
        return pl.BlockSpec(shape, lambda i: (i,) + (0,) * (nd - 1))
    return pl.BlockSpec(shape, lambda i: (0,) * len(shape))


def _front_block(body_extras, x, w, wa, i, z_refs, el_refs, er_refs,
                 mxl_refs, mxr_refs):
    valid = i * BLK + lax.broadcasted_iota(_i32, (BLK,), 0) < N
    for hh in range(HEADS):
        z = jnp.dot(x, w[hh], preferred_element_type=_f32)
        z_refs[hh][...] = z
        el = jnp.dot(z, wa[hh, :H], preferred_element_type=_f32)[:, 0]
        er = jnp.dot(z, wa[hh, H:], preferred_element_type=_f32)[:, 0]
        el_refs[hh][...] = el
        er_refs[hh][...] = er
        ml = jnp.full((16,), jnp.max(jnp.where(valid, el, -3e38)), _f32)
        mr = jnp.full((16,), jnp.max(jnp.where(valid, er, -3e38)), _f32)

        @pl.when(i == 0)
        def _():
            mxl_refs[hh][...] = ml
            mxr_refs[hh][...] = mr

        @pl.when(i > 0)
        def _():
            mxl_refs[hh][...] = jnp.maximum(mxl_refs[hh][...], ml)
            mxr_refs[hh][...] = jnp.maximum(mxr_refs[hh][...], mr)


def _tc1_body(x_ref, w_ref, wa_ref, gid_ref,
              z0_ref, z1_ref, el0_ref, el1_ref, er0_ref, er1_ref,
              mxl0_ref, mxl1_ref, mxr0_ref, mxr1_ref, msum_ref, cnt_ref):
    i = pl.program_id(0)
    x = x_ref[...]
    _front_block(None, x, w_ref[...], wa_ref[...], i,
                 (z0_ref, z1_ref), (el0_ref, el1_ref), (er0_ref, er1_ref),
                 (mxl0_ref, mxl1_ref), (mxr0_ref, mxr1_ref))
    oh = _one_hot(gid_ref[...])
    ms = lax.dot_general(oh, x, (((0,), (0,)), ((), ())),
                         preferred_element_type=_f32)
    ct = jnp.sum(oh, axis=0)[:, None]

    @pl.when(i == 0)
    def _():
        msum_ref[...] = ms
        cnt_ref[...] = ct

    @pl.when(i > 0)
    def _():
        msum_ref[...] = msum_ref[...] + ms
        cnt_ref[...] = cnt_ref[...] + ct


def _tc1_call(xp, W0, Wa0, gid):
    return pl.pallas_call(
        _tc1_body,
        grid=(NB,),
        in_specs=[_bspec((BLK, F), True), _bspec((HEADS, F, H)),
                  _bspec((HEADS, 2 * H, 1)), _bspec((BLK, 1), True)],
        out_specs=[_bspec((BLK, H), True), _bspec((BLK, H), True),
                   _bspec((BLK,), True), _bspec((BLK,), True),
                   _bspec((BLK,), True), _bspec((BLK,), True),
                   _bspec((16,)), _bspec((16,)), _bspec((16,)), _bspec((16,)),
                   _bspec((B, F)), _bspec((B, 1))],
        out_shape=[jax.ShapeDtypeStruct((NPAD, H), _f32),
                   jax.ShapeDtypeStruct((NPAD, H), _f32),
                   jax.ShapeDtypeStruct((NPAD,), _f32),
                   jax.ShapeDtypeStruct((NPAD,), _f32),
                   jax.ShapeDtypeStruct((NPAD,), _f32),
                   jax.ShapeDtypeStruct((NPAD,), _f32),
                   jax.ShapeDtypeStruct((16,), _f32),
                   jax.ShapeDtypeStruct((16,), _f32),
                   jax.ShapeDtypeStruct((16,), _f32),
                   jax.ShapeDtypeStruct((16,), _f32),
                   jax.ShapeDtypeStruct((B, F), _f32),
                   jax.ShapeDtypeStruct((B, 1), _f32)],
    )(xp, W0, Wa0, gid)


def _stats_body(hp00_ref, hp01_ref, hp10_ref, hp11_ref, sum_ref, sq_ref):
    i = pl.program_id(0)
    ss = []
    qq = []
    for hh in range(HEADS):
        hr = jnp.maximum((hp00_ref, hp10_ref)[hh][...]
                         + (hp01_ref, hp11_ref)[hh][...], 0.0)
        ss.append(jnp.sum(hr, axis=0))
        qq.append(jnp.sum(hr * hr, axis=0))
    s = jnp.stack(ss)
    q = jnp.stack(qq)

    @pl.when(i == 0)
    def _():
        sum_ref[...] = s
        sq_ref[...] = q

    @pl.when(i > 0)
    def _():
        sum_ref[...] = sum_ref[...] + s
        sq_ref[...] = sq_ref[...] + q


def _stats_call(hp):
    return pl.pallas_call(
        _stats_body,
        grid=(NB,),
        in_specs=[_bspec((BLK, H), True)] * 4,
        out_specs=[_bspec((HEADS, H)), _bspec((HEADS, H))],
        out_shape=[jax.ShapeDtypeStruct((HEADS, H), _f32),
                   jax.ShapeDtypeStruct((HEADS, H), _f32)],
    )(hp[0], hp[1], hp[2], hp[3])


def _ln_project(hp_refs, sum_, sq, gam, bet, outW, outb):
    cols = []
    for hh in range(HEADS):
        hr = jnp.maximum(hp_refs[2 * hh][...] + hp_refs[2 * hh + 1][...], 0.0)
        mean = (sum_[hh] * (1.0 / N))[None, :]
        var = (sq[hh] * (1.0 / N))[None, :] - mean * mean
        cols.append(gam[hh] * (hr - mean) * lax.rsqrt(var + 1e-5) + bet[hh])
    cat = jnp.concatenate(cols, axis=1)
    return jnp.dot(cat, outW, preferred_element_type=_f32) + outb


def _tc2_body(hp00_ref, hp01_ref, hp10_ref, hp11_ref, sum_ref, sq_ref,
              g_ref, be_ref, outW_ref, outb_ref, w_ref, wa_ref, gid_ref,
              z0_ref, z1_ref, el0_ref, el1_ref, er0_ref, er1_ref,
              mxl0_ref, mxl1_ref, mxr0_ref, mxr1_ref, msum_ref):
    i = pl.program_id(0)
    h1 = _ln_project((hp00_ref, hp01_ref, hp10_ref, hp11_ref),
                     sum_ref[...], sq_ref[...],
                     g_ref[...], be_ref[...], outW_ref[...], outb_ref[...])
    _front_block(None, h1, w_ref[...], wa_ref[...], i,
                 (z0_ref, z1_ref), (el0_ref, el1_ref), (er0_ref, er1_ref),
                 (mxl0_ref, mxl1_ref), (mxr0_ref, mxr1_ref))
    oh = _one_hot(gid_ref[...])
    ms = lax.dot_general(oh, h1, (((0,), (0,)), ((), ())),
                         preferred_element_type=_f32)

    @pl.when(i == 0)
    def _():
        msum_ref[...] = ms

    @pl.when(i > 0)
    def _():
        msum_ref[...] = msum_ref[...] + ms


def _tc2_call(hp, stats, g0, be0, out0W, out0b, W1, Wa1, gid):
    return pl.pallas_call(
        _tc2_body,
        grid=(NB,),
        in_specs=[_bspec((BLK, H), True)] * 4
        + [_bspec((HEADS, H)), _bspec((HEADS, H)),
           _bspec((HEADS, H)), _bspec((HEADS, H)),
           _bspec((HEADS * H, H)), _bspec((H,)),
           _bspec((HEADS, H, H)), _bspec((HEADS, 2 * H, 1)),
           _bspec((BLK, 1), True)],
        out_specs=[_bspec((BLK, H), True), _bspec((BLK, H), True),
                   _bspec((BLK,), True), _bspec((BLK,), True),
                   _bspec((BLK,), True), _bspec((BLK,), True),
                   _bspec((16,)), _bspec((16,)), _bspec((16,)), _bspec((16,)),
                   _bspec((B, H))],
        out_shape=[jax.ShapeDtypeStruct((NPAD, H), _f32),
                   jax.ShapeDtypeStruct((NPAD, H), _f32),
                   jax.ShapeDtypeStruct((NPAD,), _f32),
                   jax.ShapeDtypeStruct((NPAD,), _f32),
                   jax.ShapeDtypeStruct((NPAD,), _f32),
                   jax.ShapeDtypeStruct((NPAD,), _f32),
                   jax.ShapeDtypeStruct((16,), _f32),
                   jax.ShapeDtypeStruct((16,), _f32),
                   jax.ShapeDtypeStruct((16,), _f32),
                   jax.ShapeDtypeStruct((16,), _f32),
                   jax.ShapeDtypeStruct((B, H), _f32)],
    )(hp[0], hp[1], hp[2], hp[3], stats[0], stats[1],
      g0, be0, out0W, out0b, W1, Wa1, gid)


def _tc3_body(hp00_ref, hp01_ref, hp10_ref, hp11_ref, sum_ref, sq_ref,
              g_ref, be_ref, outW_ref, outb_ref, gid_ref, msum_ref):
    i = pl.program_id(0)
    h2 = _ln_project((hp00_ref, hp01_ref, hp10_ref, hp11_ref),
                     sum_ref[...], sq_ref[...],
                     g_ref[...], be_ref[...], outW_ref[...], outb_ref[...])
    oh = _one_hot(gid_ref[...])
    ms = lax.dot_general(oh, h2, (((0,), (0,)), ((), ())),
                         preferred_element_type=_f32)

    @pl.when(i == 0)
    def _():
        msum_ref[...] = ms

    @pl.when(i > 0)
    def _():
        msum_ref[...] = msum_ref[...] + ms


def _tc3_call(hp, stats, g1, be1, out1W, out1b, gid):
    return pl.pallas_call(
        _tc3_body,
        grid=(NB,),
        in_specs=[_bspec((BLK, H), True)] * 4
        + [_bspec((HEADS, H)), _bspec((HEADS, H)),
           _bspec((HEADS, H)), _bspec((HEADS, H)),
           _bspec((HEADS * H, H)), _bspec((H,)),
           _bspec((BLK, 1), True)],
        out_specs=[_bspec((B, H))],
        out_shape=[jax.ShapeDtypeStruct((B, H), _f32)],
    )(hp[0], hp[1], hp[2], hp[3], stats[0], stats[1],
      g1, be1, out1W, out1b, gid)[0]


def _lstm_layer(seq, wih, whh, bi, bh):
    hs = []
    hp = jnp.zeros((B, H), _f32)
    cp = jnp.zeros((B, H), _f32)
    for t in range(3):
        gates = (lax.dot_general(seq[t], wih, (((1,), (1,)), ((), ())),
                                 preferred_element_type=_f32) + bi
                 + lax.dot_general(hp, whh, (((1,), (1,)), ((), ())),
                                   preferred_element_type=_f32) + bh)
        i = jax.nn.sigmoid(gates[:, :H])
        f = jax.nn.sigmoid(gates[:, H:2 * H])
        g = jnp.tanh(gates[:, 2 * H:3 * H])
        o = jax.nn.sigmoid(gates[:, 3 * H:])
        cp = f * cp + i * g
        hp = o * jnp.tanh(cp)
        hs.append(hp)
    return hs, hp


def _tc4_body(msum0_ref, msum1_ref, msum2_ref, cnt_ref,
              projW_ref, projb_ref, wih_ref, whh_ref, bih_ref, bhh_ref,
              out_ref):
    cnt = jnp.maximum(cnt_ref[...], 1.0)
    m0 = msum0_ref[...] / cnt
    m1 = msum1_ref[...] / cnt
    m2 = msum2_ref[...] / cnt
    x0 = jnp.dot(m0, projW_ref[...], preferred_element_type=_f32) + projb_ref[...]
    ys, h0f = _lstm_layer([x0, m1, m2], wih_ref[0], whh_ref[0],
                          bih_ref[0], bhh_ref[0])
    _, h1f = _lstm_layer(ys, wih_ref[1], whh_ref[1], bih_ref[1], bhh_ref[1])
    out_ref[...] = h0f + h1f


def _tc4_call(msum0, msum1, msum2, cnt, projW, projb, Wih, Whh, bih, bhh):
    return pl.pallas_call(
        _tc4_body,
        out_shape=jax.ShapeDtypeStruct((B, H), _f32),
    )(msum0, msum1, msum2, cnt, projW, projb, Wih, Whh, bih, bhh)


_MESH = plsc.VectorSubcoreMesh(core_axis_name="c", subcore_axis_name="s")
_NS = 16


def _z16():
    return jnp.zeros((16,), _f32)


def _sc_a(src_hbm, dst_hbm, el0_hbm, el1_hbm, er0_hbm, er1_hbm,
          mxl0_hbm, mxl1_hbm, mxr0_hbm, mxr1_hbm, out_hbm,
          src_v, dst_v, el_v, er_v, m_v, mb_v, ex_v, z640_v, stage_v, den_sh):
    cid = lax.axis_index("c")
    sid = lax.axis_index("s")
    wid = cid * _NS + sid
    ebase = wid * EPW
    pltpu.sync_copy(src_hbm.at[pl.ds(wid * CPW, CPW)], src_v)
    pltpu.sync_copy(dst_hbm.at[pl.ds(wid * CPW, CPW)], dst_v)

    def zb(i, c):
        z640_v[pl.ds(i * 16, 16)] = _z16()
        return c
    lax.fori_loop(0, TPW // 16, zb, 0)

    for hh, (elh, erh, mlh, mrh) in enumerate(
            ((el0_hbm, er0_hbm, mxl0_hbm, mxr0_hbm),
             (el1_hbm, er1_hbm, mxl1_hbm, mxr1_hbm))):
        pltpu.sync_copy(elh, el_v)
        pltpu.sync_copy(erh, er_v)
        pltpu.sync_copy(mlh, m_v)
        pltpu.sync_copy(mrh, mb_v)
        pltpu.sync_copy(z640_v, den_sh.at[pl.ds(sid * TPW, TPW)])
        plsc.subcore_barrier()
        m16 = m_v[...] + mb_v[...]
        m16 = jnp.where(m16 >= 0.0, m16, m16 * 0.01)

        def chunk(i, c):
            def grp(g, c2):
                s16 = src_v[i, pl.ds(g * 16, 16)]
                d16 = dst_v[i, pl.ds(g * 16, 16)]
                e = plsc.load_gather(el_v, [s16]) + plsc.load_gather(er_v, [d16])
                e = jnp.where(e >= 0.0, e, e * 0.01)
                ex = jnp.exp(e - m16)
                eid = ebase + i * 128 + g * 16 + lax.iota(_i32, 16)
                ex_v[pl.ds(g * 16, 16)] = jnp.where(eid < E, ex, 0.0)
                return c2
            lax.fori_loop(0, 8, grp, 0)
            pltpu.sync_copy(ex_v, den_sh.at[dst_v.at[i]], add=True)
            return c
        lax.fori_loop(0, CPW, chunk, 0)
        plsc.subcore_barrier()
        pltpu.sync_copy(den_sh.at[pl.ds(sid * TPW, TPW)], stage_v)
        pltpu.sync_copy(stage_v,
                        out_hbm.at[pl.ds((hh * 2 + cid) * NPAD + sid * TPW, TPW)])
        plsc.subcore_barrier()


HH = H // 2


def _sc_b(src_hbm, dst_hbm, z0_hbm, z1_hbm,
          el0_hbm, el1_hbm, er0_hbm, er1_hbm,
          mxl0_hbm, mxl1_hbm, mxr0_hbm, mxr1_hbm,
          d00_hbm, d01_hbm, d10_hbm, d11_hbm, out_hbm,
          src_v, dst_v, els_v, erd_v, d0_v, d1_v, m_v, mb_v, a_v,
          rows_v, rowh_v, hout_sh, sem):
    cid = lax.axis_index("c")
    sid = lax.axis_index("s")
    wid = cid * _NS + sid
    ebase = wid * EPW
    pltpu.sync_copy(src_hbm.at[pl.ds(wid * CPW, CPW)], src_v)
    pltpu.sync_copy(dst_hbm.at[pl.ds(wid * CPW, CPW)], dst_v)

    for hh, (zh, elh, erh, mlh, mrh, d0h, d1h) in enumerate(
            ((z0_hbm, el0_hbm, er0_hbm, mxl0_hbm, mxr0_hbm, d00_hbm, d01_hbm),
             (z1_hbm, el1_hbm, er1_hbm, mxl1_hbm, mxr1_hbm, d10_hbm, d11_hbm))):
        pltpu.sync_copy(mlh, m_v)
        pltpu.sync_copy(mrh, mb_v)
        m16 = m_v[...] + mb_v[...]
        m16 = jnp.where(m16 >= 0.0, m16, m16 * 0.01)

        for half in range(2):
            def zr(i, c):
                for k in range(HH // 16):
                    rowh_v[i, pl.ds(k * 16, 16)] = _z16()
                return c
            lax.fori_loop(0, 128, zr, 0)
            for t in range(5):
                pltpu.sync_copy(rowh_v,
                                hout_sh.at[pl.ds(sid * TPW + t * 128, 128)])
            plsc.subcore_barrier()

            def chunk(i, c):
                c1 = pltpu.async_copy(zh.at[src_v.at[i]], rows_v, sem)
                c2 = pltpu.async_copy(elh.at[src_v.at[i]], els_v, sem)
                c3 = pltpu.async_copy(erh.at[dst_v.at[i]], erd_v, sem)
                c4 = pltpu.async_copy(d0h.at[dst_v.at[i]], d0_v, sem)
                c5 = pltpu.async_copy(d1h.at[dst_v.at[i]], d1_v, sem)
                c1.wait()
                c2.wait()
                c3.wait()
                c4.wait()
                c5.wait()

                def grp(g, c2_):
                    sl16 = pl.ds(g * 16, 16)
                    e = els_v[sl16] + erd_v[sl16]
                    e = jnp.where(e >= 0.0, e, e * 0.01)
                    ex = jnp.exp(e - m16)
                    eid = ebase + i * 128 + g * 16 + lax.iota(_i32, 16)
                    ex = jnp.where(eid < E, ex, 0.0)
                    dsm = d0_v[sl16] + d1_v[sl16]
                    a_v[...] = ex / jnp.maximum(dsm, 1e-16)

                    def rowscale(j, c3_):
                        sc = plsc.load_gather(a_v, [jnp.full((16,), j, _i32)])
                        r = g * 16 + j
                        for k in range(HH // 16):
                            sl = pl.ds(k * 16, 16)
                            rowh_v[r, sl] = rows_v[r, pl.ds(half * HH + k * 16, 16)] * sc
                        return c3_
                    lax.fori_loop(0, 16, rowscale, 0)
                    return c2_
                lax.fori_loop(0, 8, grp, 0)
                pltpu.sync_copy(rowh_v, hout_sh.at[dst_v.at[i]], add=True)
                return c
            lax.fori_loop(0, CPW, chunk, 0)
            plsc.subcore_barrier()
            for t in range(5):
                off = sid * TPW + t * 128
                pltpu.sync_copy(hout_sh.at[pl.ds(off, 128)], rowh_v)
                row0 = (half * HEADS * 2 + hh * 2 + cid) * NPAD + off
                pltpu.sync_copy(rowh_v, out_hbm.at[pl.ds(row0, 128)])
            plsc.subcore_barrier()


_SC_PARAMS = pltpu.CompilerParams(needs_layout_passes=False)

_sca_call = pl.kernel(
    _sc_a,
    mesh=_MESH,
    compiler_params=_SC_PARAMS,
    out_type=jax.ShapeDtypeStruct((HEADS * 2 * NPAD,), _f32),
    scratch_types=[
        pltpu.VMEM((CPW, 128), _i32),
        pltpu.VMEM((CPW, 128), _i32),
        pltpu.VMEM((NPAD,), _f32),
        pltpu.VMEM((NPAD,), _f32),
        pltpu.VMEM((16,), _f32),
        pltpu.VMEM((16,), _f32),
        pltpu.VMEM((128,), _f32),
        pltpu.VMEM((TPW,), _f32),
        pltpu.VMEM((TPW,), _f32),
        pltpu.VMEM_SHARED((NPAD,), _f32),
    ],
)

_scb_call = pl.kernel(
    _sc_b,
    mesh=_MESH,
    compiler_params=_SC_PARAMS,
    out_type=jax.ShapeDtypeStruct((2 * HEADS * 2 * NPAD, HH), _f32),
    scratch_types=[
        pltpu.VMEM((CPW, 128), _i32),
        pltpu.VMEM((CPW, 128), _i32),
        pltpu.VMEM((128,), _f32),
        pltpu.VMEM((128,), _f32),
        pltpu.VMEM((128,), _f32),
        pltpu.VMEM((128,), _f32),
        pltpu.VMEM((16,), _f32),
        pltpu.VMEM((16,), _f32),
        pltpu.VMEM((16,), _f32),
        pltpu.VMEM((128, H), _f32),
        pltpu.VMEM((128, HH), _f32),
        pltpu.VMEM_SHARED((NPAD, HH), _f32),
        pltpu.SemaphoreType.DMA,
    ],
)


def kernel(x, W0, Wa0, g0, be0, out0W, out0b, W1, Wa1, g1, be1,
           out1W, out1b, projW, projb, Wih, Whh, bih, bhh,
           edge_index, graph_ids):
    src = jnp.reshape(jnp.pad(edge_index[0], (0, EPAD - E)), (NW * CPW, 128))
    dst = jnp.reshape(jnp.pad(edge_index[1], (0, EPAD - E)), (NW * CPW, 128))
    xp = jnp.pad(x, ((0, NPAD - N), (0, 0)))
    gid = jnp.pad(graph_ids, (0, NPAD - N), constant_values=B)[:, None]

    (z0, z1, el0, el1, er0, er1, mxl0, mxl1, mxr0, mxr1,
     msum0, cnt) = _tc1_call(xp, W0, Wa0, gid)

    den0 = jnp.reshape(
        _sca_call(src, dst, el0, el1, er0, er1, mxl0, mxl1, mxr0, mxr1),
        (4, NPAD))
    hp0 = _scb_call(src, dst, z0, z1, el0, el1, er0, er1,
                    mxl0, mxl1, mxr0, mxr1,
                    den0[0], den0[1], den0[2], den0[3])
    hp0 = jnp.reshape(hp0, (2, HEADS * 2, NPAD, HH))
    hp0 = jnp.concatenate([hp0[0], hp0[1]], axis=-1)

    stats0 = _stats_call(hp0)
    (z0, z1, el0, el1, er0, er1, mxl0, mxl1, mxr0, mxr1,
     msum1) = _tc2_call(hp0, stats0, g0, be0, out0W, out0b, W1, Wa1, gid)

    den1 = jnp.reshape(
        _sca_call(src, dst, el0, el1, er0, er1, mxl0, mxl1, mxr0, mxr1),
        (4, NPAD))
    hp1 = _scb_call(src, dst, z0, z1, el0, el1, er0, er1,
                    mxl0, mxl1, mxr0, mxr1,
                    den1[0], den1[1], den1[2], den1[3])
    hp1 = jnp.reshape(hp1, (2, HEADS * 2, NPAD, HH))
    hp1 = jnp.concatenate([hp1[0], hp1[1]], axis=-1)

    stats1 = _stats_call(hp1)
    msum2 = _tc3_call(hp1, stats1, g1, be1, out1W, out1b, gid)

    return _tc4_call(msum0, msum1, msum2, cnt,
                     projW, projb, Wih, Whh, bih, bhh)


# speedup vs baseline: 15.5407x; 4.0923x over previous
import functools

import jax
import jax.numpy as jnp
from jax import lax
from jax.experimental import pallas as pl
from jax.experimental.pallas import tpu as pltpu
from jax.experimental.pallas import tpu_sc as plsc

N = 10000
E = 320000
B = 256
F = 74
H = 128
HEADS = 2

NPAD = 10240
NW = 32
CPW = 80
EPW = CPW * 128
EPAD = NW * EPW
TPW = NPAD // NW

_f32 = jnp.float32
_i32 = jnp.int32


BLK = 2048
NB = NPAD // BLK


def _one_hot(gid):
    return (gid == lax.broadcasted_iota(_i32, (1, B), 1)).astype(_f32)


def _bspec(shape, blocked_dim0=False):
    if blocked_dim0:
        nd = len(shape)
        return pl.BlockSpec(shape, lambda i: (i,) + (0,) * (nd - 1))
    return pl.BlockSpec(shape, lambda i: (0,) * len(shape))


def _front_block(body_extras, x, w, wa, i, z_refs, el_refs, er_refs,
                 mxl_refs, mxr_refs):
    valid = i * BLK + lax.broadcasted_iota(_i32, (BLK,), 0) < N
    for hh in range(HEADS):
        z = jnp.dot(x, w[hh], preferred_element_type=_f32)
        z_refs[hh][...] = z
        el = jnp.dot(z, wa[hh, :H], preferred_element_type=_f32)[:, 0]
        er = jnp.dot(z, wa[hh, H:], preferred_element_type=_f32)[:, 0]
        el_refs[hh][...] = el
        er_refs[hh][...] = er
        ml = jnp.full((16,), jnp.max(jnp.where(valid, el, -3e38)), _f32)
        mr = jnp.full((16,), jnp.max(jnp.where(valid, er, -3e38)), _f32)

        @pl.when(i == 0)
        def _():
            mxl_refs[hh][...] = ml
            mxr_refs[hh][...] = mr

        @pl.when(i > 0)
        def _():
            mxl_refs[hh][...] = jnp.maximum(mxl_refs[hh][...], ml)
            mxr_refs[hh][...] = jnp.maximum(mxr_refs[hh][...], mr)


def _tc1_body(x_ref, w_ref, wa_ref, gid_ref,
              z0_ref, z1_ref, el0_ref, el1_ref, er0_ref, er1_ref,
              mxl0_ref, mxl1_ref, mxr0_ref, mxr1_ref, msum_ref, cnt_ref):
    i = pl.program_id(0)
    x = x_ref[...]
    _front_block(None, x, w_ref[...], wa_ref[...], i,
                 (z0_ref, z1_ref), (el0_ref, el1_ref), (er0_ref, er1_ref),
                 (mxl0_ref, mxl1_ref), (mxr0_ref, mxr1_ref))
    oh = _one_hot(gid_ref[...])
    ms = lax.dot_general(oh, x, (((0,), (0,)), ((), ())),
                         preferred_element_type=_f32)
    ct = jnp.sum(oh, axis=0)[:, None]

    @pl.when(i == 0)
    def _():
        msum_ref[...] = ms
        cnt_ref[...] = ct

    @pl.when(i > 0)
    def _():
        msum_ref[...] = msum_ref[...] + ms
        cnt_ref[...] = cnt_ref[...] + ct


def _tc1_call(xp, W0, Wa0, gid):
    return pl.pallas_call(
        _tc1_body,
        grid=(NB,),
        in_specs=[_bspec((BLK, F), True), _bspec((HEADS, F, H)),
                  _bspec((HEADS, 2 * H, 1)), _bspec((BLK, 1), True)],
        out_specs=[_bspec((BLK, H), True), _bspec((BLK, H), True),
                   _bspec((BLK,), True), _bspec((BLK,), True),
                   _bspec((BLK,), True), _bspec((BLK,), True),
                   _bspec((16,)), _bspec((16,)), _bspec((16,)), _bspec((16,)),
                   _bspec((B, F)), _bspec((B, 1))],
        out_shape=[jax.ShapeDtypeStruct((NPAD, H), _f32),
                   jax.ShapeDtypeStruct((NPAD, H), _f32),
                   jax.ShapeDtypeStruct((NPAD,), _f32),
                   jax.ShapeDtypeStruct((NPAD,), _f32),
                   jax.ShapeDtypeStruct((NPAD,), _f32),
                   jax.ShapeDtypeStruct((NPAD,), _f32),
                   jax.ShapeDtypeStruct((16,), _f32),
                   jax.ShapeDtypeStruct((16,), _f32),
                   jax.ShapeDtypeStruct((16,), _f32),
                   jax.ShapeDtypeStruct((16,), _f32),
                   jax.ShapeDtypeStruct((B, F), _f32),
                   jax.ShapeDtypeStruct((B, 1), _f32)],
    )(xp, W0, Wa0, gid)


def _stats_body(hp00_ref, hp01_ref, hp10_ref, hp11_ref, sum_ref, sq_ref):
    i = pl.program_id(0)
    ss = []
    qq = []
    for hh in range(HEADS):
        hr = jnp.maximum((hp00_ref, hp10_ref)[hh][...]
                         + (hp01_ref, hp11_ref)[hh][...], 0.0)
        ss.append(jnp.sum(hr, axis=0))
        qq.append(jnp.sum(hr * hr, axis=0))
    s = jnp.stack(ss)
    q = jnp.stack(qq)

    @pl.when(i == 0)
    def _():
        sum_ref[...] = s
        sq_ref[...] = q

    @pl.when(i > 0)
    def _():
        sum_ref[...] = sum_ref[...] + s
        sq_ref[...] = sq_ref[...] + q


def _stats_call(hp):
    return pl.pallas_call(
        _stats_body,
        grid=(NB,),
        in_specs=[_bspec((BLK, H), True)] * 4,
        out_specs=[_bspec((HEADS, H)), _bspec((HEADS, H))],
        out_shape=[jax.ShapeDtypeStruct((HEADS, H), _f32),
                   jax.ShapeDtypeStruct((HEADS, H), _f32)],
    )(hp[0], hp[1], hp[2], hp[3])


def _ln_project(hp_refs, sum_, sq, gam, bet, outW, outb):
    cols = []
    for hh in range(HEADS):
        hr = jnp.maximum(hp_refs[2 * hh][...] + hp_refs[2 * hh + 1][...], 0.0)
        mean = (sum_[hh] * (1.0 / N))[None, :]
        var = (sq[hh] * (1.0 / N))[None, :] - mean * mean
        cols.append(gam[hh] * (hr - mean) * lax.rsqrt(var + 1e-5) + bet[hh])
    cat = jnp.concatenate(cols, axis=1)
    return jnp.dot(cat, outW, preferred_element_type=_f32) + outb


def _tc2_body(hp00_ref, hp01_ref, hp10_ref, hp11_ref, sum_ref, sq_ref,
              g_ref, be_ref, outW_ref, outb_ref, w_ref, wa_ref, gid_ref,
              z0_ref, z1_ref, el0_ref, el1_ref, er0_ref, er1_ref,
              mxl0_ref, mxl1_ref, mxr0_ref, mxr1_ref, msum_ref):
    i = pl.program_id(0)
    h1 = _ln_project((hp00_ref, hp01_ref, hp10_ref, hp11_ref),
                     sum_ref[...], sq_ref[...],
                     g_ref[...], be_ref[...], outW_ref[...], outb_ref[...])
    _front_block(None, h1, w_ref[...], wa_ref[...], i,
                 (z0_ref, z1_ref), (el0_ref, el1_ref), (er0_ref, er1_ref),
                 (mxl0_ref, mxl1_ref), (mxr0_ref, mxr1_ref))
    oh = _one_hot(gid_ref[...])
    ms = lax.dot_general(oh, h1, (((0,), (0,)), ((), ())),
                         preferred_element_type=_f32)

    @pl.when(i == 0)
    def _():
        msum_ref[...] = ms

    @pl.when(i > 0)
    def _():
        msum_ref[...] = msum_ref[...] + ms


def _tc2_call(hp, stats, g0, be0, out0W, out0b, W1, Wa1, gid):
    return pl.pallas_call(
        _tc2_body,
        grid=(NB,),
        in_specs=[_bspec((BLK, H), True)] * 4
        + [_bspec((HEADS, H)), _bspec((HEADS, H)),
           _bspec((HEADS, H)), _bspec((HEADS, H)),
           _bspec((HEADS * H, H)), _bspec((H,)),
           _bspec((HEADS, H, H)), _bspec((HEADS, 2 * H, 1)),
           _bspec((BLK, 1), True)],
        out_specs=[_bspec((BLK, H), True), _bspec((BLK, H), True),
                   _bspec((BLK,), True), _bspec((BLK,), True),
                   _bspec((BLK,), True), _bspec((BLK,), True),
                   _bspec((16,)), _bspec((16,)), _bspec((16,)), _bspec((16,)),
                   _bspec((B, H))],
        out_shape=[jax.ShapeDtypeStruct((NPAD, H), _f32),
                   jax.ShapeDtypeStruct((NPAD, H), _f32),
                   jax.ShapeDtypeStruct((NPAD,), _f32),
                   jax.ShapeDtypeStruct((NPAD,), _f32),
                   jax.ShapeDtypeStruct((NPAD,), _f32),
                   jax.ShapeDtypeStruct((NPAD,), _f32),
                   jax.ShapeDtypeStruct((16,), _f32),
                   jax.ShapeDtypeStruct((16,), _f32),
                   jax.ShapeDtypeStruct((16,), _f32),
                   jax.ShapeDtypeStruct((16,), _f32),
                   jax.ShapeDtypeStruct((B, H), _f32)],
    )(hp[0], hp[1], hp[2], hp[3], stats[0], stats[1],
      g0, be0, out0W, out0b, W1, Wa1, gid)


def _tc3_body(hp00_ref, hp01_ref, hp10_ref, hp11_ref, sum_ref, sq_ref,
              g_ref, be_ref, outW_ref, outb_ref, gid_ref, msum_ref):
    i = pl.program_id(0)
    h2 = _ln_project((hp00_ref, hp01_ref, hp10_ref, hp11_ref),
                     sum_ref[...], sq_ref[...],
                     g_ref[...], be_ref[...], outW_ref[...], outb_ref[...])
    oh = _one_hot(gid_ref[...])
    ms = lax.dot_general(oh, h2, (((0,), (0,)), ((), ())),
                         preferred_element_type=_f32)

    @pl.when(i == 0)
    def _():
        msum_ref[...] = ms

    @pl.when(i > 0)
    def _():
        msum_ref[...] = msum_ref[...] + ms


def _tc3_call(hp, stats, g1, be1, out1W, out1b, gid):
    return pl.pallas_call(
        _tc3_body,
        grid=(NB,),
        in_specs=[_bspec((BLK, H), True)] * 4
        + [_bspec((HEADS, H)), _bspec((HEADS, H)),
           _bspec((HEADS, H)), _bspec((HEADS, H)),
           _bspec((HEADS * H, H)), _bspec((H,)),
           _bspec((BLK, 1), True)],
        out_specs=[_bspec((B, H))],
        out_shape=[jax.ShapeDtypeStruct((B, H), _f32)],
    )(hp[0], hp[1], hp[2], hp[3], stats[0], stats[1],
      g1, be1, out1W, out1b, gid)[0]


def _lstm_layer(seq, wih, whh, bi, bh):
    hs = []
    hp = jnp.zeros((B, H), _f32)
    cp = jnp.zeros((B, H), _f32)
    for t in range(3):
        gates = (lax.dot_general(seq[t], wih, (((1,), (1,)), ((), ())),
                                 preferred_element_type=_f32) + bi
                 + lax.dot_general(hp, whh, (((1,), (1,)), ((), ())),
                                   preferred_element_type=_f32) + bh)
        i = jax.nn.sigmoid(gates[:, :H])
        f = jax.nn.sigmoid(gates[:, H:2 * H])
        g = jnp.tanh(gates[:, 2 * H:3 * H])
        o = jax.nn.sigmoid(gates[:, 3 * H:])
        cp = f * cp + i * g
        hp = o * jnp.tanh(cp)
        hs.append(hp)
    return hs, hp


def _tc4_body(msum0_ref, msum1_ref, msum2_ref, cnt_ref,
              projW_ref, projb_ref, wih_ref, whh_ref, bih_ref, bhh_ref,
              out_ref):
    cnt = jnp.maximum(cnt_ref[...], 1.0)
    m0 = msum0_ref[...] / cnt
    m1 = msum1_ref[...] / cnt
    m2 = msum2_ref[...] / cnt
    x0 = jnp.dot(m0, projW_ref[...], preferred_element_type=_f32) + projb_ref[...]
    ys, h0f = _lstm_layer([x0, m1, m2], wih_ref[0], whh_ref[0],
                          bih_ref[0], bhh_ref[0])
    _, h1f = _lstm_layer(ys, wih_ref[1], whh_ref[1], bih_ref[1], bhh_ref[1])
    out_ref[...] = h0f + h1f


def _tc4_call(msum0, msum1, msum2, cnt, projW, projb, Wih, Whh, bih, bhh):
    return pl.pallas_call(
        _tc4_body,
        out_shape=jax.ShapeDtypeStruct((B, H), _f32),
    )(msum0, msum1, msum2, cnt, projW, projb, Wih, Whh, bih, bhh)


_MESH = plsc.VectorSubcoreMesh(core_axis_name="c", subcore_axis_name="s")
_NS = 16


def _z16():
    return jnp.zeros((16,), _f32)


def _sc_a(src_hbm, dst_hbm, el0_hbm, el1_hbm, er0_hbm, er1_hbm,
          mxl0_hbm, mxl1_hbm, mxr0_hbm, mxr1_hbm, out_hbm,
          src_v, dst_v, el_v, er_v, m_v, mb_v, ex_v, z640_v, stage_v, den_sh):
    cid = lax.axis_index("c")
    sid = lax.axis_index("s")
    wid = cid * _NS + sid
    ebase = wid * EPW
    pltpu.sync_copy(src_hbm.at[pl.ds(wid * CPW, CPW)], src_v)
    pltpu.sync_copy(dst_hbm.at[pl.ds(wid * CPW, CPW)], dst_v)

    def zb(i, c):
        z640_v[pl.ds(i * 16, 16)] = _z16()
        return c
    lax.fori_loop(0, TPW // 16, zb, 0)

    for hh, (elh, erh, mlh, mrh) in enumerate(
            ((el0_hbm, er0_hbm, mxl0_hbm, mxr0_hbm),
             (el1_hbm, er1_hbm, mxl1_hbm, mxr1_hbm))):
        pltpu.sync_copy(elh, el_v)
        pltpu.sync_copy(erh, er_v)
        pltpu.sync_copy(mlh, m_v)
        pltpu.sync_copy(mrh, mb_v)
        pltpu.sync_copy(z640_v, den_sh.at[pl.ds(sid * TPW, TPW)])
        plsc.subcore_barrier()
        m16 = m_v[...] + mb_v[...]
        m16 = jnp.where(m16 >= 0.0, m16, m16 * 0.01)

        def chunk(i, c):
            def grp(g, c2):
                s16 = src_v[i, pl.ds(g * 16, 16)]
                d16 = dst_v[i, pl.ds(g * 16, 16)]
                e = plsc.load_gather(el_v, [s16]) + plsc.load_gather(er_v, [d16])
                e = jnp.where(e >= 0.0, e, e * 0.01)
                ex = jnp.exp(e - m16)
                eid = ebase + i * 128 + g * 16 + lax.iota(_i32, 16)
                ex_v[pl.ds(g * 16, 16)] = jnp.where(eid < E, ex, 0.0)
                return c2
            lax.fori_loop(0, 8, grp, 0)
            pltpu.sync_copy(ex_v, den_sh.at[dst_v.at[i]], add=True)
            return c
        lax.fori_loop(0, CPW, chunk, 0)
        plsc.subcore_barrier()
        pltpu.sync_copy(den_sh.at[pl.ds(sid * TPW, TPW)], stage_v)
        pltpu.sync_copy(stage_v,
                        out_hbm.at[pl.ds((hh * 2 + cid) * NPAD + sid * TPW, TPW)])
        plsc.subcore_barrier()


def _sc_a2(src_hbm, dst_hbm, el0_hbm, el1_hbm, er0_hbm, er1_hbm,
           mxl0_hbm, mxl1_hbm, mxr0_hbm, mxr1_hbm, den_hbm, out_hbm,
           src_v, dst_v, el_v, er_v, ds_v, tmp_v, m_v, mb_v, a8_v):
    cid = lax.axis_index("c")
    sid = lax.axis_index("s")
    wid = cid * _NS + sid
    ebase = wid * EPW
    pltpu.sync_copy(src_hbm.at[pl.ds(wid * CPW, CPW)], src_v)
    pltpu.sync_copy(dst_hbm.at[pl.ds(wid * CPW, CPW)], dst_v)

    for hh, (elh, erh, mlh, mrh) in enumerate(
            ((el0_hbm, er0_hbm, mxl0_hbm, mxr0_hbm),
             (el1_hbm, er1_hbm, mxl1_hbm, mxr1_hbm))):
        pltpu.sync_copy(elh, el_v)
        pltpu.sync_copy(erh, er_v)
        pltpu.sync_copy(mlh, m_v)
        pltpu.sync_copy(mrh, mb_v)
        pltpu.sync_copy(den_hbm.at[pl.ds(2 * hh * NPAD, NPAD)], ds_v)
        pltpu.sync_copy(den_hbm.at[pl.ds((2 * hh + 1) * NPAD, NPAD)], tmp_v)

        def dsum(i, c):
            sl = pl.ds(i * 16, 16)
            ds_v[sl] = ds_v[sl] + tmp_v[sl]
            return c
        lax.fori_loop(0, NPAD // 16, dsum, 0)
        m16 = m_v[...] + mb_v[...]
        m16 = jnp.where(m16 >= 0.0, m16, m16 * 0.01)

        def blk(b, c):
            def chunk(j, c1):
                i = b * 8 + j

                def grp(g, c2):
                    s16 = src_v[i, pl.ds(g * 16, 16)]
                    d16 = dst_v[i, pl.ds(g * 16, 16)]
                    e = (plsc.load_gather(el_v, [s16])
                         + plsc.load_gather(er_v, [d16]))
                    e = jnp.where(e >= 0.0, e, e * 0.01)
                    ex = jnp.exp(e - m16)
                    eid = ebase + i * 128 + g * 16 + lax.iota(_i32, 16)
                    ex = jnp.where(eid < E, ex, 0.0)
                    dsm = plsc.load_gather(ds_v, [d16])
                    a8_v[j, pl.ds(g * 16, 16)] = ex / jnp.maximum(dsm, 1e-16)
                    return c2
                lax.fori_loop(0, 8, grp, 0)
                return c1
            lax.fori_loop(0, 8, chunk, 0)
            row0 = (hh * NW + wid) * CPW + b * 8
            pltpu.sync_copy(a8_v, out_hbm.at[pl.ds(row0, 8)])
            return c
        lax.fori_loop(0, CPW // 8, blk, 0)


def _sc_b(src_hbm, dst_hbm, z0_hbm, z1_hbm, a_hbm, out_hbm,
          si_v, di_v, av_v, a_v, rows_a, rows_b, hout_sh, gsa, gsb):
    cid = lax.axis_index("c")
    sid = lax.axis_index("s")
    wid = cid * _NS + sid

    for hh, zh in enumerate((z0_hbm, z1_hbm)):
        def zr(i, c):
            for k in range(H // 16):
                rows_a[i, pl.ds(k * 16, 16)] = _z16()
            return c
        lax.fori_loop(0, 128, zr, 0)
        for t in range(5):
            pltpu.sync_copy(rows_a, hout_sh.at[pl.ds(sid * TPW + t * 128, 128)])
        plsc.subcore_barrier()

        def proc(cb, rows, sem):
            sem.wait()

            def grp(g, c2):
                a_v[...] = av_v[cb, pl.ds(g * 16, 16)]

                def rowscale(j, c3):
                    sc = plsc.load_gather(a_v, [jnp.full((16,), j, _i32)])
                    r = g * 16 + j
                    for k in range(H // 16):
                        sl = pl.ds(k * 16, 16)
                        rows[r, sl] = rows[r, sl] * sc
                    return c3
                lax.fori_loop(0, 16, rowscale, 0)
                return c2
            lax.fori_loop(0, 8, grp, 0)
            pltpu.sync_copy(rows, hout_sh.at[di_v.at[cb]], add=True)

        def fire(cb, rows, gsem):
            return pltpu.async_copy(zh.at[si_v.at[cb]], rows, gsem)

        def blk(b, c):
            base = wid * CPW + b * 8
            pltpu.sync_copy(src_hbm.at[pl.ds(base, 8)], si_v)
            pltpu.sync_copy(dst_hbm.at[pl.ds(base, 8)], di_v)
            pltpu.sync_copy(a_hbm.at[pl.ds(hh * NW * CPW + base, 8)], av_v)
            def seq(j, c1):
                proc(j, rows_a, fire(j, rows_a, gsa))
                return c1
            lax.fori_loop(0, 8, seq, 0)
            return c
        lax.fori_loop(0, CPW // 8, blk, 0)
        plsc.subcore_barrier()
        for t in range(5):
            off = sid * TPW + t * 128
            pltpu.sync_copy(hout_sh.at[pl.ds(off, 128)], rows_a)
            row0 = (hh * 2 + cid) * NPAD + off
            pltpu.sync_copy(rows_a, out_hbm.at[pl.ds(row0, 128)])
        plsc.subcore_barrier()


_SC_PARAMS = pltpu.CompilerParams(needs_layout_passes=False)

_sca_call = pl.kernel(
    _sc_a,
    mesh=_MESH,
    compiler_params=_SC_PARAMS,
    out_type=jax.ShapeDtypeStruct((HEADS * 2 * NPAD,), _f32),
    scratch_types=[
        pltpu.VMEM((CPW, 128), _i32),
        pltpu.VMEM((CPW, 128), _i32),
        pltpu.VMEM((NPAD,), _f32),
        pltpu.VMEM((NPAD,), _f32),
        pltpu.VMEM((16,), _f32),
        pltpu.VMEM((16,), _f32),
        pltpu.VMEM((128,), _f32),
        pltpu.VMEM((TPW,), _f32),
        pltpu.VMEM((TPW,), _f32),
        pltpu.VMEM_SHARED((NPAD,), _f32),
    ],
)

_sca2_call = pl.kernel(
    _sc_a2,
    mesh=_MESH,
    compiler_params=_SC_PARAMS,
    out_type=jax.ShapeDtypeStruct((HEADS * NW * CPW, 128), _f32),
    scratch_types=[
        pltpu.VMEM((CPW, 128), _i32),
        pltpu.VMEM((CPW, 128), _i32),
        pltpu.VMEM((NPAD,), _f32),
        pltpu.VMEM((NPAD,), _f32),
        pltpu.VMEM((NPAD,), _f32),
        pltpu.VMEM((NPAD,), _f32),
        pltpu.VMEM((16,), _f32),
        pltpu.VMEM((16,), _f32),
        pltpu.VMEM((8, 128), _f32),
    ],
)

_scb_call = pl.kernel(
    _sc_b,
    mesh=_MESH,
    compiler_params=_SC_PARAMS,
    out_type=jax.ShapeDtypeStruct((HEADS * 2 * NPAD, H), _f32),
    scratch_types=[
        pltpu.VMEM((8, 128), _i32),
        pltpu.VMEM((8, 128), _i32),
        pltpu.VMEM((8, 128), _f32),
        pltpu.VMEM((16,), _f32),
        pltpu.VMEM((128, H), _f32),
        pltpu.VMEM((128, H), _f32),
        pltpu.VMEM_SHARED((NPAD, H), _f32),
        pltpu.SemaphoreType.DMA,
        pltpu.SemaphoreType.DMA,
    ],
)


def kernel(x, W0, Wa0, g0, be0, out0W, out0b, W1, Wa1, g1, be1,
           out1W, out1b, projW, projb, Wih, Whh, bih, bhh,
           edge_index, graph_ids):
    pad_idx = jnp.arange(EPAD - E, dtype=_i32) * 7 % N
    src = jnp.reshape(jnp.concatenate([edge_index[0], pad_idx]), (NW * CPW, 128))
    dst = jnp.reshape(jnp.concatenate([edge_index[1], pad_idx]), (NW * CPW, 128))
    xp = jnp.pad(x, ((0, NPAD - N), (0, 0)))
    gid = jnp.pad(graph_ids, (0, NPAD - N), constant_values=B)[:, None]

    (z0, z1, el0, el1, er0, er1, mxl0, mxl1, mxr0, mxr1,
     msum0, cnt) = _tc1_call(xp, W0, Wa0, gid)

    den0 = _sca_call(src, dst, el0, el1, er0, er1, mxl0, mxl1, mxr0, mxr1)
    aw0 = _sca2_call(src, dst, el0, el1, er0, er1, mxl0, mxl1, mxr0, mxr1,
                     den0)
    hp0 = jnp.reshape(_scb_call(src, dst, z0, z1, aw0),
                      (HEADS * 2, NPAD, H))

    stats0 = _stats_call(hp0)
    (z0, z1, el0, el1, er0, er1, mxl0, mxl1, mxr0, mxr1,
     msum1) = _tc2_call(hp0, stats0, g0, be0, out0W, out0b, W1, Wa1, gid)

    den1 = _sca_call(src, dst, el0, el1, er0, er1, mxl0, mxl1, mxr0, mxr1)
    aw1 = _sca2_call(src, dst, el0, el1, er0, er1, mxl0, mxl1, mxr0, mxr1,
                     den1)
    hp1 = jnp.reshape(_scb_call(src, dst, z0, z1, aw1),
                      (HEADS * 2, NPAD, H))

    stats1 = _stats_call(hp1)
    msum2 = _tc3_call(hp1, stats1, g1, be1, out1W, out1b, gid)

    return _tc4_call(msum0, msum1, msum2, cnt,
                     projW, projb, Wih, Whh, bih, bhh)
